# Initial kernel scaffold; baseline (speedup 1.0000x reference)
import jax
import jax.numpy as jnp
from jax.experimental import pallas as pl


def kernel(x, edge_index, edge_attr, W0, W1, b):
    raise NotImplementedError("write your pallas kernel here")


# speedup vs baseline: 5.2769x; 5.2769x over previous
import functools

import jax
import jax.numpy as jnp
from jax import lax
from jax.experimental import pallas as pl
from jax.experimental.pallas import tpu as pltpu
from jax.experimental.pallas import tpu_sc as plsc

N = 2048
E = 32768
LANES = 16
NCORES = 2
NSUB = 16
CHUNK_ROWS = 512
CHUNKS = 2
EDGES_PER_TEC = E // NSUB
ROWS_PER_TEC = CHUNK_ROWS // NSUB
SCAT = 128
ROUNDS = EDGES_PER_TEC // SCAT
GROUPS = SCAT // LANES
ZLEN = 16384


def _build_adj_body(dst_hbm, src_hbm, w_hbm, a_hbm,
                    acc_sh, ebuf_d, ebuf_s, ebuf_w, idx_b, val_b, zbuf):
    c = lax.axis_index("c")
    s = lax.axis_index("s")

    ebase = s * EDGES_PER_TEC
    pltpu.sync_copy(dst_hbm.at[pl.ds(ebase, EDGES_PER_TEC)], ebuf_d)
    pltpu.sync_copy(src_hbm.at[pl.ds(ebase, EDGES_PER_TEC)], ebuf_s)
    pltpu.sync_copy(w_hbm.at[pl.ds(ebase, EDGES_PER_TEC)], ebuf_w)

    def _zb(i, carry):
        zbuf[pl.ds(i * LANES, LANES)] = jnp.zeros((LANES,), jnp.float32)
        return carry
    lax.fori_loop(0, ZLEN // LANES, _zb, 0)

    my_words = ROWS_PER_TEC * N

    for chunk in range(CHUNKS):
        row_base = c * (CHUNK_ROWS * CHUNKS) + chunk * CHUNK_ROWS

        for z in range(my_words // ZLEN):
            pltpu.sync_copy(zbuf, acc_sh.at[pl.ds(s * my_words + z * ZLEN, ZLEN)])
        plsc.subcore_barrier()

        def _round(r, carry):
            def _group(g, carry2):
                e0 = (r * SCAT) + g * LANES
                d = ebuf_d[pl.ds(e0, LANES)]
                sv = ebuf_s[pl.ds(e0, LANES)]
                wv = ebuf_w[pl.ds(e0, LANES)]
                dl = d - row_base
                m = (dl >= 0) & (dl < CHUNK_ROWS)
                idx_b[pl.ds(g * LANES, LANES)] = jnp.where(m, dl * N + sv, 0)
                val_b[pl.ds(g * LANES, LANES)] = jnp.where(m, wv, 0.0)
                return carry2
            lax.fori_loop(0, GROUPS, _group, 0)
            pltpu.sync_copy(val_b, acc_sh.at[idx_b], add=True)
            return carry
        lax.fori_loop(0, ROUNDS, _round, 0)
        def _zfill(g, carry):
            idx_b[pl.ds(g * LANES, LANES)] = jnp.zeros((LANES,), jnp.int32)
            val_b[pl.ds(g * LANES, LANES)] = jnp.zeros((LANES,), jnp.float32)
            return carry
        lax.fori_loop(0, GROUPS, _zfill, 0)
        pltpu.sync_copy(val_b, acc_sh.at[idx_b], add=True)
        pltpu.sync_copy(val_b, acc_sh.at[idx_b], add=True)
        plsc.subcore_barrier()

        pltpu.sync_copy(
            acc_sh.at[pl.ds(s * my_words, my_words)],
            a_hbm.at[pl.ds(row_base * N + s * my_words, my_words)])
        plsc.subcore_barrier()


def _build_adj(dst, src, w):
    run = pl.kernel(
        _build_adj_body,
        out_type=jax.ShapeDtypeStruct((N * N,), jnp.float32),
        mesh=plsc.VectorSubcoreMesh(core_axis_name="c", subcore_axis_name="s"),
        scratch_types=[
            pltpu.VMEM_SHARED((CHUNK_ROWS * N,), jnp.float32),
            pltpu.VMEM((EDGES_PER_TEC,), jnp.int32),
            pltpu.VMEM((EDGES_PER_TEC,), jnp.int32),
            pltpu.VMEM((EDGES_PER_TEC,), jnp.float32),
            pltpu.VMEM((SCAT,), jnp.int32),
            pltpu.VMEM((SCAT,), jnp.float32),
            pltpu.VMEM((ZLEN,), jnp.float32),
        ],
    )
    return run(dst, src, w)


BM = 512
BN = 512
BK = 512


def _mm_nt_body(x_ref, w_ref, o_ref):
    o_ref[...] = lax.dot_general(
        x_ref[...], w_ref[...], (((1,), (1,)), ((), ())),
        preferred_element_type=jnp.float32)


def _dis_body(a_ref, o_ref):
    deg = jnp.sum(a_ref[...], axis=0, keepdims=True)
    safe = jnp.where(deg > 0, deg, 1.0)
    o_ref[...] = jnp.where(deg > 0, lax.rsqrt(safe), 0.0)


def _fused_body(x_ref, w0_ref, a_ref, g_ref, disk_ref, disi_ref, b_ref,
                o_ref, acc0, acc1):
    k = pl.program_id(2)

    @pl.when(k == 0)
    def _init():
        acc0[...] = jnp.zeros_like(acc0)
        acc1[...] = jnp.zeros_like(acc1)

    acc0[...] += lax.dot_general(
        x_ref[...], w0_ref[...], (((1,), (1,)), ((), ())),
        preferred_element_type=jnp.float32)
    gs = g_ref[...] * disk_ref[...]
    acc1[...] += lax.dot_general(
        a_ref[...], gs, (((1,), (0,)), ((), ())),
        preferred_element_type=jnp.float32)

    @pl.when(k == pl.num_programs(2) - 1)
    def _fin():
        o_ref[...] = jnp.maximum(
            acc0[...] - disi_ref[...] * acc1[...] + b_ref[...], 0.0)


def _mm_nt(x, w):
    return pl.pallas_call(
        _mm_nt_body,
        out_shape=jax.ShapeDtypeStruct((N, N), jnp.float32),
        grid=(N // BM, N // BN),
        in_specs=[
            pl.BlockSpec((BM, N), lambda i, j: (i, 0)),
            pl.BlockSpec((BN, N), lambda i, j: (j, 0)),
        ],
        out_specs=pl.BlockSpec((BM, BN), lambda i, j: (i, j)),
        compiler_params=pltpu.CompilerParams(
            dimension_semantics=("parallel", "parallel")),
    )(x, w)


def _dis_from_adj(a):
    return pl.pallas_call(
        _dis_body,
        out_shape=jax.ShapeDtypeStruct((1, N), jnp.float32),
        grid=(N // BN,),
        in_specs=[pl.BlockSpec((N, BN), lambda j: (0, j))],
        out_specs=pl.BlockSpec((1, BN), lambda j: (0, j)),
        compiler_params=pltpu.CompilerParams(
            dimension_semantics=("parallel",)),
    )(a)


def _fused_out(x, w0, a, g, dis_col, b2d):
    nk = N // BK
    return pl.pallas_call(
        _fused_body,
        out_shape=jax.ShapeDtypeStruct((N, N), jnp.float32),
        grid=(N // BM, N // BN, nk),
        in_specs=[
            pl.BlockSpec((BM, BK), lambda i, j, k: (i, k)),
            pl.BlockSpec((BN, BK), lambda i, j, k: (j, k)),
            pl.BlockSpec((BM, BK), lambda i, j, k: (i, k)),
            pl.BlockSpec((BK, BN), lambda i, j, k: (k, j)),
            pl.BlockSpec((BK, 1), lambda i, j, k: (k, 0)),
            pl.BlockSpec((BM, 1), lambda i, j, k: (i, 0)),
            pl.BlockSpec((1, BN), lambda i, j, k: (0, j)),
        ],
        out_specs=pl.BlockSpec((BM, BN), lambda i, j, k: (i, j)),
        scratch_shapes=[
            pltpu.VMEM((BM, BN), jnp.float32),
            pltpu.VMEM((BM, BN), jnp.float32),
        ],
        compiler_params=pltpu.CompilerParams(
            dimension_semantics=("parallel", "parallel", "arbitrary")),
    )(x, w0, a, g, dis_col, dis_col, b2d)


def kernel(x, edge_index, edge_attr, W0, W1, b):
    X = x[:, 0, :]
    src = edge_index[0]
    dst = edge_index[1]

    a_flat = _build_adj(dst, src, edge_attr)
    g = _mm_nt(X, W1)
    a = a_flat.reshape(N, N)
    dis_row = _dis_from_adj(a)
    dis_col = dis_row.reshape(N, 1)
    out = _fused_out(X, W0, a, g, dis_col, b.reshape(1, N))
    return out[None, :, :]


# speedup vs baseline: 5.9306x; 1.1239x over previous
import functools

import jax
import jax.numpy as jnp
from jax import lax
from jax.experimental import pallas as pl
from jax.experimental.pallas import tpu as pltpu
from jax.experimental.pallas import tpu_sc as plsc

N = 2048
E = 32768
LANES = 16
NCORES = 2
NSUB = 16
CHUNK_ROWS = 512
CHUNKS = 2
EDGES_PER_TEC = E // NSUB
ROWS_PER_TEC = CHUNK_ROWS // NSUB
SCAT = 128
ROUNDS = EDGES_PER_TEC // SCAT
GROUPS = SCAT // LANES
ZLEN = 16384


def _build_adj_body(dst_hbm, src_hbm, w_hbm, a_hbm,
                    acc_sh, ebuf_d, ebuf_s, ebuf_w, idx_b, val_b, zbuf, sem):
    c = lax.axis_index("c")
    s = lax.axis_index("s")

    ebase = s * EDGES_PER_TEC
    e0d = pltpu.async_copy(dst_hbm.at[pl.ds(ebase, EDGES_PER_TEC)], ebuf_d, sem)
    e0s = pltpu.async_copy(src_hbm.at[pl.ds(ebase, EDGES_PER_TEC)], ebuf_s, sem)
    e0w = pltpu.async_copy(w_hbm.at[pl.ds(ebase, EDGES_PER_TEC)], ebuf_w, sem)

    def _zb(i, carry):
        zbuf[pl.ds(i * LANES, LANES)] = jnp.zeros((LANES,), jnp.float32)
        return carry
    lax.fori_loop(0, ZLEN // LANES, _zb, 0)
    e0d.wait(); e0s.wait(); e0w.wait()

    my_words = ROWS_PER_TEC * N

    for chunk in range(CHUNKS):
        row_base = c * (CHUNK_ROWS * CHUNKS) + chunk * CHUNK_ROWS

        zcopies = [
            pltpu.async_copy(
                zbuf, acc_sh.at[pl.ds(s * my_words + z * ZLEN, ZLEN)], sem)
            for z in range(my_words // ZLEN)
        ]
        def _round(r, carry):
            def _group(g, carry2):
                e0 = (r * SCAT) + g * LANES
                d = ebuf_d[pl.ds(e0, LANES)]
                sv = ebuf_s[pl.ds(e0, LANES)]
                wv = ebuf_w[pl.ds(e0, LANES)]
                dl = d - row_base
                m = (dl >= 0) & (dl < CHUNK_ROWS)
                idx_b[r, pl.ds(g * LANES, LANES)] = jnp.where(m, dl * N + sv, 0)
                val_b[r, pl.ds(g * LANES, LANES)] = jnp.where(m, wv, 0.0)
                return carry2
            lax.fori_loop(0, GROUPS, _group, 0)
            return carry
        lax.fori_loop(0, ROUNDS, _round, 0)
        for zc in zcopies:
            zc.wait()
        plsc.subcore_barrier()

        scats = [
            pltpu.async_copy(val_b.at[r], acc_sh.at[idx_b.at[r]], sem, add=True)
            for r in range(ROUNDS)
        ]
        for sc in scats:
            sc.wait()
        def _zfill(g, carry):
            idx_b[0, pl.ds(g * LANES, LANES)] = jnp.zeros((LANES,), jnp.int32)
            val_b[0, pl.ds(g * LANES, LANES)] = jnp.zeros((LANES,), jnp.float32)
            return carry
        lax.fori_loop(0, GROUPS, _zfill, 0)
        pltpu.sync_copy(val_b.at[0], acc_sh.at[idx_b.at[0]], add=True)
        pltpu.sync_copy(val_b.at[0], acc_sh.at[idx_b.at[0]], add=True)
        plsc.subcore_barrier()

        pltpu.sync_copy(
            acc_sh.at[pl.ds(s * my_words, my_words)],
            a_hbm.at[pl.ds(row_base * N + s * my_words, my_words)])
        plsc.subcore_barrier()


def _build_adj(dst, src, w):
    run = pl.kernel(
        _build_adj_body,
        out_type=jax.ShapeDtypeStruct((N * N,), jnp.float32),
        mesh=plsc.VectorSubcoreMesh(core_axis_name="c", subcore_axis_name="s"),
        scratch_types=[
            pltpu.VMEM_SHARED((CHUNK_ROWS * N,), jnp.float32),
            pltpu.VMEM((EDGES_PER_TEC,), jnp.int32),
            pltpu.VMEM((EDGES_PER_TEC,), jnp.int32),
            pltpu.VMEM((EDGES_PER_TEC,), jnp.float32),
            pltpu.VMEM((ROUNDS, SCAT), jnp.int32),
            pltpu.VMEM((ROUNDS, SCAT), jnp.float32),
            pltpu.VMEM((ZLEN,), jnp.float32),
            pltpu.SemaphoreType.DMA,
        ],
    )
    return run(dst, src, w)


BM = 512
BN = 512
BK = 512


def _mm2_body(x_ref, w0_ref, w1_ref, m1_ref, g_ref):
    m1_ref[...] = lax.dot_general(
        x_ref[...], w0_ref[...], (((1,), (1,)), ((), ())),
        preferred_element_type=jnp.float32)
    g_ref[...] = lax.dot_general(
        x_ref[...], w1_ref[...], (((1,), (1,)), ((), ())),
        preferred_element_type=jnp.float32)


def _dis_body(a_ref, o_ref):
    deg = jnp.sum(a_ref[...], axis=0, keepdims=True)
    safe = jnp.where(deg > 0, deg, 1.0)
    o_ref[...] = jnp.where(deg > 0, lax.rsqrt(safe), 0.0)


def _fused_body(m1_ref, a_ref, g_ref, disk_ref, disi_ref, b_ref,
                o_ref, acc1):
    k = pl.program_id(2)

    @pl.when(k == 0)
    def _init():
        acc1[...] = jnp.zeros_like(acc1)

    gs = g_ref[...] * disk_ref[...]
    acc1[...] += lax.dot_general(
        a_ref[...], gs, (((1,), (0,)), ((), ())),
        preferred_element_type=jnp.float32)

    @pl.when(k == pl.num_programs(2) - 1)
    def _fin():
        o_ref[...] = jnp.maximum(
            m1_ref[...] - disi_ref[...] * acc1[...] + b_ref[...], 0.0)


def _mm2(x, w0, w1):
    return pl.pallas_call(
        _mm2_body,
        out_shape=(jax.ShapeDtypeStruct((N, N), jnp.float32),
                   jax.ShapeDtypeStruct((N, N), jnp.float32)),
        grid=(N // BM, N // BN),
        in_specs=[
            pl.BlockSpec((BM, N), lambda i, j: (i, 0)),
            pl.BlockSpec((BN, N), lambda i, j: (j, 0)),
            pl.BlockSpec((BN, N), lambda i, j: (j, 0)),
        ],
        out_specs=(pl.BlockSpec((BM, BN), lambda i, j: (i, j)),
                   pl.BlockSpec((BM, BN), lambda i, j: (i, j))),
        compiler_params=pltpu.CompilerParams(
            dimension_semantics=("parallel", "parallel")),
    )(x, w0, w1)


def _dis_from_adj(a):
    return pl.pallas_call(
        _dis_body,
        out_shape=jax.ShapeDtypeStruct((1, N), jnp.float32),
        grid=(N // BN,),
        in_specs=[pl.BlockSpec((N, BN), lambda j: (0, j))],
        out_specs=pl.BlockSpec((1, BN), lambda j: (0, j)),
        compiler_params=pltpu.CompilerParams(
            dimension_semantics=("parallel",)),
    )(a)


def _fused_out(m1, a, g, dis_col, b2d):
    nk = N // BK
    return pl.pallas_call(
        _fused_body,
        out_shape=jax.ShapeDtypeStruct((N, N), jnp.float32),
        grid=(N // BM, N // BN, nk),
        in_specs=[
            pl.BlockSpec((BM, BN), lambda i, j, k: (i, j)),
            pl.BlockSpec((BM, BK), lambda i, j, k: (i, k)),
            pl.BlockSpec((BK, BN), lambda i, j, k: (k, j)),
            pl.BlockSpec((BK, 1), lambda i, j, k: (k, 0)),
            pl.BlockSpec((BM, 1), lambda i, j, k: (i, 0)),
            pl.BlockSpec((1, BN), lambda i, j, k: (0, j)),
        ],
        out_specs=pl.BlockSpec((BM, BN), lambda i, j, k: (i, j)),
        scratch_shapes=[
            pltpu.VMEM((BM, BN), jnp.float32),
        ],
        compiler_params=pltpu.CompilerParams(
            dimension_semantics=("parallel", "parallel", "arbitrary")),
    )(m1, a, g, dis_col, dis_col, b2d)


def kernel(x, edge_index, edge_attr, W0, W1, b):
    X = x[:, 0, :]
    src = edge_index[0]
    dst = edge_index[1]

    a_flat = _build_adj(dst, src, edge_attr)
    m1, g = _mm2(X, W0, W1)
    a = a_flat.reshape(N, N)
    dis_row = _dis_from_adj(a)
    dis_col = dis_row.reshape(N, 1)
    out = _fused_out(m1, a, g, dis_col, b.reshape(1, N))
    return out[None, :, :]


# speedup vs baseline: 7.6253x; 1.2858x over previous
import functools

import jax
import jax.numpy as jnp
from jax import lax
from jax.experimental import pallas as pl
from jax.experimental.pallas import tpu as pltpu
from jax.experimental.pallas import tpu_sc as plsc

N = 2048
E = 32768
LANES = 16
NCORES = 2
NSUB = 16
CHUNK_ROWS = 512
CHUNKS = 2
EDGES_PER_TEC = E // NSUB
ROWS_PER_TEC = CHUNK_ROWS // NSUB
SCAT = 128
ROUNDS = EDGES_PER_TEC // SCAT
GROUPS = SCAT // LANES
ZLEN = 16384


def _build_adj_body(dst_hbm, src_hbm, w_hbm, a_hbm, deg_hbm,
                    acc_sh, deg_sh, ebuf_d, ebuf_s, ebuf_w, idx_b, val_b,
                    zbuf, sem):
    c = lax.axis_index("c")
    s = lax.axis_index("s")

    rbase = s * ROUNDS
    e0d = pltpu.async_copy(dst_hbm.at[pl.ds(rbase, ROUNDS)], ebuf_d, sem)
    e0s = pltpu.async_copy(src_hbm.at[pl.ds(rbase, ROUNDS)], ebuf_s, sem)
    e0w = pltpu.async_copy(w_hbm.at[pl.ds(rbase, ROUNDS)], ebuf_w, sem)

    def _zb(i, carry):
        zbuf[pl.ds(i * LANES, LANES)] = jnp.zeros((LANES,), jnp.float32)
        return carry
    lax.fori_loop(0, ZLEN // LANES, _zb, 0)
    e0d.wait(); e0s.wait(); e0w.wait()

    my_words = ROWS_PER_TEC * N

    for chunk in range(CHUNKS):
        row_base = c * (CHUNK_ROWS * CHUNKS) + chunk * CHUNK_ROWS

        zcs = [pltpu.async_copy(
            zbuf, acc_sh.at[pl.ds(s * my_words + z * ZLEN, ZLEN)], sem)
            for z in range(my_words // ZLEN)]
        if chunk == 0:
            @pl.when(c == 0)
            def _zdeg():
                pltpu.sync_copy(zbuf.at[pl.ds(0, N // NSUB)],
                                deg_sh.at[pl.ds(s * (N // NSUB), N // NSUB)])

        def _round(r, carry):
            def _group(g, carry2):
                d = ebuf_d[r, pl.ds(g * LANES, LANES)]
                sv = ebuf_s[r, pl.ds(g * LANES, LANES)]
                wv = ebuf_w[r, pl.ds(g * LANES, LANES)]
                dl = d - row_base
                m = (dl >= 0) & (dl < CHUNK_ROWS)
                idx_b[r, pl.ds(g * LANES, LANES)] = jnp.where(m, dl * N + sv, 0)
                val_b[r, pl.ds(g * LANES, LANES)] = jnp.where(m, wv, 0.0)
                return carry2
            lax.fori_loop(0, GROUPS, _group, 0)
            return carry
        lax.fori_loop(0, ROUNDS, _round, 0)
        for zc in zcs:
            zc.wait()
        plsc.subcore_barrier()

        scats = [
            pltpu.async_copy(val_b.at[r], acc_sh.at[idx_b.at[r]], sem, add=True)
            for r in range(ROUNDS)
        ]
        if chunk == 0:
            @pl.when(c == 0)
            def _degscat():
                for r in range(ROUNDS):
                    pltpu.sync_copy(ebuf_w.at[r], deg_sh.at[ebuf_s.at[r]],
                                    add=True)
        for scd in scats:
            scd.wait()
        def _zfill(g, carry):
            idx_b[0, pl.ds(g * LANES, LANES)] = jnp.zeros((LANES,), jnp.int32)
            val_b[0, pl.ds(g * LANES, LANES)] = jnp.zeros((LANES,), jnp.float32)
            return carry
        lax.fori_loop(0, GROUPS, _zfill, 0)
        pltpu.sync_copy(val_b.at[0], acc_sh.at[idx_b.at[0]], add=True)
        pltpu.sync_copy(val_b.at[0], acc_sh.at[idx_b.at[0]], add=True)
        plsc.subcore_barrier()

        pltpu.sync_copy(
            acc_sh.at[pl.ds(s * my_words, my_words)],
            a_hbm.at[pl.ds(row_base * N + s * my_words, my_words)])
        if chunk == 0:
            @pl.when(c == 0)
            def _degout():
                pltpu.sync_copy(deg_sh.at[pl.ds(s * (N // NSUB), N // NSUB)],
                                deg_hbm.at[pl.ds(s * (N // NSUB), N // NSUB)])
        plsc.subcore_barrier()


def _build_adj(dst2d, src2d, w2d):
    run = pl.kernel(
        _build_adj_body,
        out_type=(jax.ShapeDtypeStruct((N * N,), jnp.float32),
                  jax.ShapeDtypeStruct((N,), jnp.float32)),
        mesh=plsc.VectorSubcoreMesh(core_axis_name="c", subcore_axis_name="s"),
        scratch_types=[
            pltpu.VMEM_SHARED((CHUNK_ROWS * N,), jnp.float32),
            pltpu.VMEM_SHARED((N,), jnp.float32),
            pltpu.VMEM((ROUNDS, SCAT), jnp.int32),
            pltpu.VMEM((ROUNDS, SCAT), jnp.int32),
            pltpu.VMEM((ROUNDS, SCAT), jnp.float32),
            pltpu.VMEM((ROUNDS, SCAT), jnp.int32),
            pltpu.VMEM((ROUNDS, SCAT), jnp.float32),
            pltpu.VMEM((ZLEN,), jnp.float32),
            pltpu.SemaphoreType.DMA,
        ],
    )
    return run(dst2d, src2d, w2d)


BM = 1024
BN = 1024
BK = 512
MM = 1024
MN = 512


def _mm2_body(x_ref, w0_ref, w1_ref, m1_ref, g_ref):
    m1_ref[...] = lax.dot_general(
        x_ref[...], w0_ref[...], (((1,), (1,)), ((), ())),
        preferred_element_type=jnp.float32)
    g_ref[...] = lax.dot_general(
        x_ref[...], w1_ref[...], (((1,), (1,)), ((), ())),
        preferred_element_type=jnp.float32)


def _dis(deg):
    safe = jnp.where(deg > 0, deg, 1.0)
    return jnp.where(deg > 0, lax.rsqrt(safe), 0.0)


def _fused_body(m1_ref, a_ref, g_ref, degk_ref, degi_ref, b_ref,
                o_ref, acc1):
    k = pl.program_id(2)

    @pl.when(k == 0)
    def _init():
        acc1[...] = jnp.zeros_like(acc1)

    gs = g_ref[...] * _dis(degk_ref[...])
    acc1[...] += lax.dot_general(
        a_ref[...], gs, (((1,), (0,)), ((), ())),
        preferred_element_type=jnp.float32)

    @pl.when(k == pl.num_programs(2) - 1)
    def _fin():
        o_ref[...] = jnp.maximum(
            m1_ref[...] - _dis(degi_ref[...]) * acc1[...] + b_ref[...], 0.0)


def _mm2(x, w0, w1):
    return pl.pallas_call(
        _mm2_body,
        out_shape=(jax.ShapeDtypeStruct((N, N), jnp.float32),
                   jax.ShapeDtypeStruct((N, N), jnp.float32)),
        grid=(N // MM, N // MN),
        in_specs=[
            pl.BlockSpec((MM, N), lambda i, j: (i, 0)),
            pl.BlockSpec((MN, N), lambda i, j: (j, 0)),
            pl.BlockSpec((MN, N), lambda i, j: (j, 0)),
        ],
        out_specs=(pl.BlockSpec((MM, MN), lambda i, j: (i, j)),
                   pl.BlockSpec((MM, MN), lambda i, j: (i, j))),
        compiler_params=pltpu.CompilerParams(
            dimension_semantics=("parallel", "parallel")),
    )(x, w0, w1)


def _fused_out(m1, a, g, deg_col, b2d):
    nk = N // BK
    return pl.pallas_call(
        _fused_body,
        out_shape=jax.ShapeDtypeStruct((N, N), jnp.float32),
        grid=(N // BM, N // BN, nk),
        in_specs=[
            pl.BlockSpec((BM, BN), lambda i, j, k: (i, j)),
            pl.BlockSpec((BM, BK), lambda i, j, k: (i, k)),
            pl.BlockSpec((BK, BN), lambda i, j, k: (k, j)),
            pl.BlockSpec((BK, 1), lambda i, j, k: (k, 0)),
            pl.BlockSpec((BM, 1), lambda i, j, k: (i, 0)),
            pl.BlockSpec((1, BN), lambda i, j, k: (0, j)),
        ],
        out_specs=pl.BlockSpec((BM, BN), lambda i, j, k: (i, j)),
        scratch_shapes=[
            pltpu.VMEM((BM, BN), jnp.float32),
        ],
        compiler_params=pltpu.CompilerParams(
            dimension_semantics=("parallel", "parallel", "arbitrary")),
    )(m1, a, g, deg_col, deg_col, b2d)


def kernel(x, edge_index, edge_attr, W0, W1, b):
    X = x[:, 0, :]
    src = edge_index[0].reshape(E // SCAT, SCAT)
    dst = edge_index[1].reshape(E // SCAT, SCAT)
    w2d = edge_attr.reshape(E // SCAT, SCAT)

    a_flat, deg = _build_adj(dst, src, w2d)
    m1, g = _mm2(X, W0, W1)
    a = a_flat.reshape(N, N)
    out = _fused_out(m1, a, g, deg.reshape(N, 1), b.reshape(1, N))
    return out[None, :, :]
